# Initial kernel scaffold; baseline (speedup 1.0000x reference)
import jax
import jax.numpy as jnp
from jax.experimental import pallas as pl


def kernel(x, ln_g, ln_b, W1, b1, W2, b2, W3, b3, W4, b4):
    raise NotImplementedError("write your pallas kernel here")


# speedup vs baseline: 1.1276x; 1.1276x over previous
import functools

import jax
import jax.numpy as jnp
from jax import lax
from jax.experimental import pallas as pl
from jax.experimental.pallas import tpu as pltpu
from jax.experimental.pallas import tpu_sc as plsc

B, T, N, C = 16, 16, 196, 768
E = 2 * C
K = 8
S = 24
SEG = N * C // S
SPW = K * S // 2
RPD = 8
NDMA = SPW // RPD
NW = 32


def _gelu_exact(v):
    return 0.5 * v * (1.0 + lax.erf(v * 0.7071067811865476))


def _tc_body(x_ref, lng_ref, lnb_ref, w1_ref, b1_ref, w2_ref, b2_ref,
             w3_ref, b3_ref, w4_ref, b4_ref, out_ref, pooled_ref):
    bI = pl.program_id(0)
    tI = pl.program_id(1)
    xb = x_ref[0, 0]
    s = jnp.sum(xb, axis=0, keepdims=True) * (1.0 / N)
    m = jnp.max(xb, axis=0, keepdims=True)
    row = bI * T + tI
    pooled_ref[pl.ds(row, 1), 0:C] = s
    pooled_ref[pl.ds(row, 1), C:E] = m

    @pl.when((bI == B - 1) & (tI == T - 1))
    def _():
        p = pooled_ref[...]
        mu = jnp.mean(p, axis=-1, keepdims=True)
        var = jnp.mean((p - mu) ** 2, axis=-1, keepdims=True)
        xn = (p - mu) * lax.rsqrt(var + 1e-5) * lng_ref[...] + lnb_ref[...]
        h1 = _gelu_exact(
            jnp.dot(xn, w1_ref[...], preferred_element_type=jnp.float32)
            + b1_ref[...])
        local = h1[:, :C]
        glob = h1[:, C:].reshape(B, T, C)
        gm = jnp.mean(glob, axis=1, keepdims=True)
        gmb = jnp.broadcast_to(gm, (B, T, C)).reshape(B * T, C)
        h = jnp.concatenate([local, gmb], axis=-1)
        h2 = _gelu_exact(
            jnp.dot(h, w2_ref[...], preferred_element_type=jnp.float32)
            + b2_ref[...])
        h3 = _gelu_exact(
            jnp.dot(h2, w3_ref[...], preferred_element_type=jnp.float32)
            + b3_ref[...])
        sc = jnp.sum(h3 * w4_ref[...], axis=-1, keepdims=True) + b4_ref[...]
        out_ref[...] = jnp.broadcast_to(sc, (B * T, 128))


def _tc_scores(x4, lng, lnb, W1, b1, W2, b2, W3, b3, w4row, b4v):
    full = lambda shape: pl.BlockSpec(shape, lambda b, t: tuple(0 for _ in shape))
    return pl.pallas_call(
        _tc_body,
        grid=(B, T),
        in_specs=[
            pl.BlockSpec((1, 1, N, C), lambda b, t: (b, t, 0, 0)),
            full((1, E)), full((1, E)),
            full((E, E)), full((1, E)),
            full((E, C)), full((1, C)),
            full((C, C // 2)), full((1, C // 2)),
            full((1, C // 2)), full((1, 1)),
        ],
        out_specs=pl.BlockSpec((B * T, 128), lambda b, t: (0, 0)),
        out_shape=jax.ShapeDtypeStruct((B * T, 128), jnp.float32),
        scratch_shapes=[pltpu.VMEM((B * T, E), jnp.float32)],
    )(x4, lng, lnb, W1, b1, W2, b2, W3, b3, w4row, b4v)


def _sc_body(scores_hbm, x_hbm, out_hbm, score_v, ids_v, idx_v,
             buf0, buf1, sem0, sem1):
    cid = lax.axis_index("c")
    sid = lax.axis_index("s")
    wid = sid * 2 + cid
    b = wid // 2
    h = wid % 2

    pltpu.sync_copy(scores_hbm.at[pl.ds(b * T, T)], score_v)
    sv = score_v[...]
    lanes = lax.iota(jnp.int32, 16)
    _, by_score = plsc.sort_key_val(sv, lanes, descending=True)
    selk = jnp.where(lanes < K, by_score, jnp.int32(2147483647))
    sids, _ = plsc.sort_key_val(selk, selk)
    ids_v[...] = sids

    for i in range(SPW // 16):
        g = h * SPW + i * 16 + lanes
        kk = g // S
        ss = g % S
        idk = plsc.load_gather(ids_v, [kk])
        idx_v[pl.ds(i * 16, 16)] = (b * T + idk) * S + ss

    base_out = b * (K * S) + h * SPW
    bufs = (buf0, buf1)
    sems = (sem0, sem1)
    cps = [None, None]
    cps[0] = pltpu.async_copy(
        x_hbm.at[idx_v.at[pl.ds(0, RPD)]], buf0, sem0)
    for j in range(1, NDMA + 1):
        if j < NDMA:
            cps[j % 2] = pltpu.async_copy(
                x_hbm.at[idx_v.at[pl.ds(j * RPD, RPD)]],
                bufs[j % 2], sems[j % 2])
        cps[(j - 1) % 2].wait()
        pltpu.sync_copy(bufs[(j - 1) % 2],
                        out_hbm.at[pl.ds(base_out + (j - 1) * RPD, RPD)])


@functools.cache
def _sc_gather():
    return pl.kernel(
        _sc_body,
        out_type=jax.ShapeDtypeStruct((B * K * S, SEG), jnp.float32),
        mesh=plsc.VectorSubcoreMesh(core_axis_name="c", subcore_axis_name="s"),
        compiler_params=pltpu.CompilerParams(needs_layout_passes=False),
        scratch_types=[
            pltpu.VMEM((T,), jnp.float32),
            pltpu.VMEM((16,), jnp.int32),
            pltpu.VMEM((SPW,), jnp.int32),
            pltpu.VMEM((RPD, SEG), jnp.float32),
            pltpu.VMEM((RPD, SEG), jnp.float32),
            pltpu.SemaphoreType.DMA,
            pltpu.SemaphoreType.DMA,
        ],
    )


def kernel(x, ln_g, ln_b, W1, b1, W2, b2, W3, b3, W4, b4):
    x4 = x.reshape(B, T, N, C)
    scores_pad = _tc_scores(
        x4,
        ln_g.reshape(1, E), ln_b.reshape(1, E),
        W1, b1.reshape(1, E),
        W2, b2.reshape(1, C),
        W3, b3.reshape(1, C // 2),
        W4.reshape(1, C // 2), b4.reshape(1, 1),
    )
    scores = scores_pad[:, 0]
    x_tab = x.reshape(B * T * S, SEG)
    out_tab = _sc_gather()(scores, x_tab)
    return out_tab.reshape(B, K * N, C)


# speedup vs baseline: 1.5893x; 1.4095x over previous
import functools

import jax
import jax.numpy as jnp
from jax import lax
from jax.experimental import pallas as pl
from jax.experimental.pallas import tpu as pltpu
from jax.experimental.pallas import tpu_sc as plsc

B, T, N, C = 16, 16, 196, 768
E = 2 * C
K = 8
SPP = K * N // 2
RPD = 56
NDMA = SPP // RPD
NW = 32


def _gelu_exact(v):
    return 0.5 * v * (1.0 + lax.erf(v * 0.7071067811865476))


def _tc_body(x_ref, lng_ref, lnb_ref, w1_ref, b1_ref, w2_ref, b2_ref,
             w3_ref, b3_ref, w4_ref, b4_ref, out_ref, pooled_ref):
    bI = pl.program_id(0)
    tI = pl.program_id(1)
    xb = x_ref[0, 0]
    s = jnp.sum(xb, axis=0, keepdims=True) * (1.0 / N)
    m = jnp.max(xb, axis=0, keepdims=True)
    row = bI * T + tI
    pooled_ref[pl.ds(row, 1), 0:C] = s
    pooled_ref[pl.ds(row, 1), C:E] = m

    @pl.when((bI == B - 1) & (tI == T - 1))
    def _():
        p = pooled_ref[...]
        mu = jnp.mean(p, axis=-1, keepdims=True)
        var = jnp.mean((p - mu) ** 2, axis=-1, keepdims=True)
        xn = (p - mu) * lax.rsqrt(var + 1e-5) * lng_ref[...] + lnb_ref[...]
        h1 = _gelu_exact(
            jnp.dot(xn, w1_ref[...], preferred_element_type=jnp.float32)
            + b1_ref[...])
        local = h1[:, :C]
        glob = h1[:, C:].reshape(B, T, C)
        gm = jnp.mean(glob, axis=1, keepdims=True)
        gmb = jnp.broadcast_to(gm, (B, T, C)).reshape(B * T, C)
        h = jnp.concatenate([local, gmb], axis=-1)
        h2 = _gelu_exact(
            jnp.dot(h, w2_ref[...], preferred_element_type=jnp.float32)
            + b2_ref[...])
        h3 = _gelu_exact(
            jnp.dot(h2, w3_ref[...], preferred_element_type=jnp.float32)
            + b3_ref[...])
        sc = jnp.sum(h3 * w4_ref[...], axis=-1, keepdims=True) + b4_ref[...]
        out_ref[...] = jnp.broadcast_to(sc, (B * T, 128))


def _tc_scores(x4, lng, lnb, W1, b1, W2, b2, W3, b3, w4row, b4v):
    full = lambda shape: pl.BlockSpec(shape, lambda b, t: tuple(0 for _ in shape))
    return pl.pallas_call(
        _tc_body,
        grid=(B, T),
        in_specs=[
            pl.BlockSpec((1, 1, N, C), lambda b, t: (b, t, 0, 0)),
            full((1, E)), full((1, E)),
            full((E, E)), full((1, E)),
            full((E, C)), full((1, C)),
            full((C, C // 2)), full((1, C // 2)),
            full((1, C // 2)), full((1, 1)),
        ],
        out_specs=pl.BlockSpec((B * T, 128), lambda b, t: (0, 0)),
        out_shape=jax.ShapeDtypeStruct((B * T, 128), jnp.float32),
        scratch_shapes=[pltpu.VMEM((B * T, E), jnp.float32)],
    )(x4, lng, lnb, W1, b1, W2, b2, W3, b3, w4row, b4v)


def _sc_body(scores_hbm, x_hbm, out_hbm, score_v, ids_v, idx_v,
             buf0, buf1, sem0, sem1):
    cid = lax.axis_index("c")
    sid = lax.axis_index("s")
    wid = sid * 2 + cid
    b = wid // 2
    h = wid % 2

    pltpu.sync_copy(scores_hbm.at[pl.ds(b * T, T)], score_v)
    sv = score_v[...]
    lanes = lax.iota(jnp.int32, 16)
    _, by_score = plsc.sort_key_val(sv, lanes, descending=True)
    selk = jnp.where(lanes < K, by_score, jnp.int32(2147483647))
    sids, _ = plsc.sort_key_val(selk, selk)
    ids_v[...] = sids

    for i in range(SPP // 16):
        g = h * SPP + i * 16 + lanes
        kk = g // N
        rr = g % N
        idk = plsc.load_gather(ids_v, [kk])
        idx_v[pl.ds(i * 16, 16)] = (b * T + idk) * N + rr

    base_out = b * (K * N) + h * SPP
    bufs = (buf0, buf1)
    sems = (sem0, sem1)
    cps = [None, None]
    cps[0] = pltpu.async_copy(
        x_hbm.at[idx_v.at[pl.ds(0, RPD)]], buf0, sem0)
    for j in range(1, NDMA + 1):
        if j < NDMA:
            cps[j % 2] = pltpu.async_copy(
                x_hbm.at[idx_v.at[pl.ds(j * RPD, RPD)]],
                bufs[j % 2], sems[j % 2])
        cps[(j - 1) % 2].wait()
        pltpu.sync_copy(bufs[(j - 1) % 2],
                        out_hbm.at[pl.ds(base_out + (j - 1) * RPD, RPD)])


@functools.cache
def _sc_gather():
    return pl.kernel(
        _sc_body,
        out_type=jax.ShapeDtypeStruct((B * K * N, C), jnp.float32),
        mesh=plsc.VectorSubcoreMesh(core_axis_name="c", subcore_axis_name="s"),
        compiler_params=pltpu.CompilerParams(needs_layout_passes=False),
        scratch_types=[
            pltpu.VMEM((T,), jnp.float32),
            pltpu.VMEM((16,), jnp.int32),
            pltpu.VMEM((SPP,), jnp.int32),
            pltpu.VMEM((RPD, C), jnp.float32),
            pltpu.VMEM((RPD, C), jnp.float32),
            pltpu.SemaphoreType.DMA,
            pltpu.SemaphoreType.DMA,
        ],
    )


def kernel(x, ln_g, ln_b, W1, b1, W2, b2, W3, b3, W4, b4):
    x4 = x.reshape(B, T, N, C)
    scores_pad = _tc_scores(
        x4,
        ln_g.reshape(1, E), ln_b.reshape(1, E),
        W1, b1.reshape(1, E),
        W2, b2.reshape(1, C),
        W3, b3.reshape(1, C // 2),
        W4.reshape(1, C // 2), b4.reshape(1, 1),
    )
    scores = scores_pad[:, 0]
    x_tab = x.reshape(B * T * N, C)
    out_tab = _sc_gather()(scores, x_tab)
    return out_tab.reshape(B, K * N, C)


# speedup vs baseline: 3.6547x; 2.2995x over previous
import functools

import numpy as np
import jax
import jax.numpy as jnp
from jax import lax
from jax.experimental import pallas as pl
from jax.experimental.pallas import tpu as pltpu
from jax.experimental.pallas import tpu_sc as plsc

B, T, N, C = 16, 16, 196, 768
E = 2 * C
K = 8
SPP = K * N // 2
RPD = 56
NDMA = SPP // RPD
NW = 32


_PREC = None

_ERFC_P = [2.326819970068386E-2, -1.387039388740657E-1, 3.687424674597105E-1,
           -5.824733027278666E-1, 6.210004621745983E-1, -4.944515323274145E-1,
           3.404879937665872E-1, -2.741127028184656E-1, 5.638259427386472E-1]
_ERFC_R = [-1.047766399936249E+1, 1.297719955372516E+1, -7.495518717768503E+0,
           2.921019019210786E+0, -1.015265279202700E+0, 4.218463358204948E-1,
           -2.820767439740514E-1, 5.641895067754075E-1]
_ERF_T = [7.853861353153693E-5, -8.010193625184903E-4, 5.188327685732524E-3,
          -2.685381193529856E-2, 1.128358514861418E-1, -3.761262582423300E-1,
          1.128379165726710E+0]
_SQRT_HALF = float(np.float32(0.7071067811865476))


def _poly(y, cs):
    p = jnp.full_like(y, np.float32(cs[0]))
    for c in cs[1:]:
        p = p * y + np.float32(c)
    return p


def _erfc_f32(u):
    au = jnp.abs(u)
    nx2 = -u * u
    z = jnp.exp(nx2)
    q = 1.0 / au
    y = q * q
    p = jnp.where(au < 2.0, _poly(y, _ERFC_P), _poly(y, _ERFC_R))
    yi = z * q * p
    yi = jnp.where(nx2 < -88.72283905206835, jnp.zeros_like(yi), yi)
    yi = jnp.where(u < 0.0, 2.0 - yi, yi)
    es = u * _poly(u * u, _ERF_T)
    return jnp.where(au > 1.0, yi, 1.0 - es)


def _gelu_exact(v):
    return 0.5 * v * _erfc_f32(-v * _SQRT_HALF)


def _tc_body(x_ref, lng_ref, lnb_ref, w1_ref, b1_ref, w2_ref, b2_ref,
             w3_ref, b3_ref, w4_ref, b4_ref, out_ref, pooled_ref):
    bI = pl.program_id(0)
    tI = pl.program_id(1)
    x2f = x_ref[0]
    for f in range(2):
        xb = x2f[f * N:(f + 1) * N]
        s = jnp.sum(xb, axis=0, keepdims=True) * (1.0 / N)
        m = jnp.max(xb, axis=0, keepdims=True)
        row = bI * T + tI * 2 + f
        pooled_ref[pl.ds(row, 1), 0:C] = s
        pooled_ref[pl.ds(row, 1), C:E] = m

    @pl.when((bI == B - 1) & (tI == T // 2 - 1))
    def _():
        p = pooled_ref[...]
        mu = jnp.mean(p, axis=-1, keepdims=True)
        var = jnp.mean((p - mu) ** 2, axis=-1, keepdims=True)
        xn = (p - mu) / jnp.sqrt(var + 1e-5) * lng_ref[...] + lnb_ref[...]
        h1 = _gelu_exact(
            jnp.dot(xn, w1_ref[...], preferred_element_type=jnp.float32,
                    precision=_PREC)
            + b1_ref[...])
        local = h1[:, :C]
        glob = h1[:, C:].reshape(B, T, C)
        gm = jnp.mean(glob, axis=1, keepdims=True)
        gmb = jnp.broadcast_to(gm, (B, T, C)).reshape(B * T, C)
        h = jnp.concatenate([local, gmb], axis=-1)
        h2 = _gelu_exact(
            jnp.dot(h, w2_ref[...], preferred_element_type=jnp.float32,
                    precision=_PREC)
            + b2_ref[...])
        h3 = _gelu_exact(
            jnp.dot(h2, w3_ref[...], preferred_element_type=jnp.float32,
                    precision=_PREC)
            + b3_ref[...])
        w4m = jnp.broadcast_to(w4_ref[...].reshape(C // 2, 1), (C // 2, 128))
        sc = jnp.dot(h3, w4m, preferred_element_type=jnp.float32,
                     precision=_PREC) + b4_ref[...]
        g3 = sc.reshape(B, T, 128)
        smin = jnp.min(g3, axis=1, keepdims=True)
        smax = jnp.max(g3, axis=1, keepdims=True)
        nrm = (g3 - smin) / (smax - smin + 1e-5)
        out_ref[...] = nrm.reshape(B * T, 128)


def _tc_scores(x4, lng, lnb, W1, b1, W2, b2, W3, b3, w4row, b4v):
    full = lambda shape: pl.BlockSpec(shape, lambda b, t: tuple(0 for _ in shape))
    return pl.pallas_call(
        _tc_body,
        grid=(B, T // 2),
        in_specs=[
            pl.BlockSpec((1, 2 * N, C), lambda b, t: (b, t, 0)),
            full((1, E)), full((1, E)),
            full((E, E)), full((1, E)),
            full((E, C)), full((1, C)),
            full((C, C // 2)), full((1, C // 2)),
            full((1, C // 2)), full((1, 1)),
        ],
        out_specs=pl.BlockSpec((B * T, 128), lambda b, t: (0, 0)),
        out_shape=jax.ShapeDtypeStruct((B * T, 128), jnp.float32),
        scratch_shapes=[pltpu.VMEM((B * T, E), jnp.float32)],
    )(x4, lng, lnb, W1, b1, W2, b2, W3, b3, w4row, b4v)


def _sc_body(scores_hbm, x_hbm, out_hbm, score_v, ids_v,
             idx0, idx1, idx2, idx3, idx4, idx5, idx6,
             buf0, buf1, sem0, sem1):
    cid = lax.axis_index("c")
    sid = lax.axis_index("s")
    wid = sid * 2 + cid
    b = wid // 2
    h = wid % 2

    pltpu.sync_copy(scores_hbm.at[pl.ds(b * T, T)], score_v)
    sv = score_v[...]
    lanes = lax.iota(jnp.int32, 16)
    _, by_score = plsc.sort_key_val(sv, lanes, descending=True)
    selk = jnp.where(lanes < K, by_score, jnp.int32(2147483647))
    sids, _ = plsc.sort_key_val(selk, selk)
    ids_v[...] = sids

    idxs = (idx0, idx1, idx2, idx3, idx4, idx5, idx6)
    for i in range(SPP // 16):
        g = h * SPP + i * 16 + lanes
        kk = g // N
        rr = g % N
        idk = plsc.load_gather(ids_v, [kk])
        idxs[i // 7][pl.ds((i % 7) * 16, 16)] = (b * T + idk) * N + rr

    base_out = b * (K * N) + h * SPP
    bufs = (buf0, buf1)
    sems = (sem0, sem1)

    def _src(j):
        return x_hbm.at[idxs[j // 2].at[pl.ds((j % 2) * RPD, RPD)]]

    cps = [None, None]
    cps[0] = pltpu.async_copy(_src(0), buf0, sem0)
    for j in range(1, NDMA + 1):
        if j < NDMA:
            cps[j % 2] = pltpu.async_copy(_src(j), bufs[j % 2], sems[j % 2])
        cps[(j - 1) % 2].wait()
        pltpu.sync_copy(bufs[(j - 1) % 2],
                        out_hbm.at[pl.ds(base_out + (j - 1) * RPD, RPD)])


@functools.cache
def _sc_gather():
    return pl.kernel(
        _sc_body,
        out_type=jax.ShapeDtypeStruct((B * K * N, C), jnp.float32),
        mesh=plsc.VectorSubcoreMesh(core_axis_name="c", subcore_axis_name="s"),
        compiler_params=pltpu.CompilerParams(needs_layout_passes=False),
        scratch_types=[
            pltpu.VMEM((T,), jnp.float32),
            pltpu.VMEM((16,), jnp.int32),
            pltpu.VMEM((112,), jnp.int32),
            pltpu.VMEM((112,), jnp.int32),
            pltpu.VMEM((112,), jnp.int32),
            pltpu.VMEM((112,), jnp.int32),
            pltpu.VMEM((112,), jnp.int32),
            pltpu.VMEM((112,), jnp.int32),
            pltpu.VMEM((112,), jnp.int32),
            pltpu.VMEM((RPD, C), jnp.float32),
            pltpu.VMEM((RPD, C), jnp.float32),
            pltpu.SemaphoreType.DMA,
            pltpu.SemaphoreType.DMA,
        ],
    )


def kernel(x, ln_g, ln_b, W1, b1, W2, b2, W3, b3, W4, b4):
    scores_pad = _tc_scores(
        x,
        ln_g.reshape(1, E), ln_b.reshape(1, E),
        W1, b1.reshape(1, E),
        W2, b2.reshape(1, C),
        W3, b3.reshape(1, C // 2),
        W4.reshape(1, C // 2), b4.reshape(1, 1),
    )
    scores = scores_pad[:, 0]
    x_tab = x.reshape(B * T * N, C)
    out_tab = _sc_gather()(scores, x_tab)
    return out_tab.reshape(B, K * N, C)
